# baseline (speedup 1.0000x reference)
import jax
import jax.numpy as jnp
from jax.experimental import pallas as pl
from jax.experimental.pallas import tpu as pltpu

N = 10000
F = 128
C = 64
ROWS = 200


def _spmm1_kernel(adj_ref, x_ref, w_in_ref, b_in_ref, b0_ref, w1_ref, o_ref):
    a = adj_ref[...]
    ax = jnp.dot(a, x_ref[...], preferred_element_type=jnp.float32)
    rs = jnp.sum(a, axis=1, keepdims=True)
    h = jnp.maximum(
        jnp.dot(ax, w_in_ref[...], preferred_element_type=jnp.float32)
        + rs * b_in_ref[...]
        + b0_ref[...],
        0.0,
    )
    o_ref[...] = jnp.dot(h, w1_ref[...], preferred_element_type=jnp.float32)


def _spmm2_kernel(adj_ref, t_ref, b_ref, w_ref, b2_ref, o_ref):
    acc = jnp.dot(adj_ref[...], t_ref[...], preferred_element_type=jnp.float32)
    h = jnp.maximum(acc + b_ref[...], 0.0)
    o_ref[...] = (
        jnp.dot(h, w_ref[...], preferred_element_type=jnp.float32) + b2_ref[...]
    )


def _spmm1(adj, x, w_in, b_in, b0, w1):
    grid = (N // ROWS,)
    return pl.pallas_call(
        _spmm1_kernel,
        grid=grid,
        in_specs=[
            pl.BlockSpec((ROWS, N), lambda i: (i, 0)),
            pl.BlockSpec((N, F), lambda i: (0, 0)),
            pl.BlockSpec((F, F), lambda i: (0, 0)),
            pl.BlockSpec((1, F), lambda i: (0, 0)),
            pl.BlockSpec((1, F), lambda i: (0, 0)),
            pl.BlockSpec((F, F), lambda i: (0, 0)),
        ],
        out_specs=pl.BlockSpec((ROWS, F), lambda i: (i, 0)),
        out_shape=jax.ShapeDtypeStruct((N, F), jnp.float32),
        compiler_params=pltpu.CompilerParams(
            dimension_semantics=("parallel",)
        ),
    )(adj, x, w_in, b_in, b0, w1)


def _spmm2(adj, t, b, w, b2):
    grid = (N // ROWS,)
    return pl.pallas_call(
        _spmm2_kernel,
        grid=grid,
        in_specs=[
            pl.BlockSpec((ROWS, N), lambda i: (i, 0)),
            pl.BlockSpec((N, F), lambda i: (0, 0)),
            pl.BlockSpec((1, F), lambda i: (0, 0)),
            pl.BlockSpec((F, C), lambda i: (0, 0)),
            pl.BlockSpec((1, C), lambda i: (0, 0)),
        ],
        out_specs=pl.BlockSpec((ROWS, C), lambda i: (i, 0)),
        out_shape=jax.ShapeDtypeStruct((N, C), jnp.float32),
        compiler_params=pltpu.CompilerParams(
            dimension_semantics=("parallel",)
        ),
    )(adj, t, b, w, b2)


def kernel(x, adj, bn_gamma, bn_beta, fc_in_w, fc_in_b,
           gc0_w, gc0_b, gc1_w, gc1_b, fc_out_w, fc_out_b):
    eps = 1e-5
    scale = bn_gamma / jnp.sqrt(1.0 + eps)
    w_in = (scale[:, None] * fc_in_w.T) @ gc0_w
    b_in = ((bn_beta @ fc_in_w.T + fc_in_b) @ gc0_w)[None, :]

    t1 = _spmm1(adj, x, w_in, b_in, gc0_b, gc1_w)
    out = _spmm2(adj, t1, gc1_b, fc_out_w.T, fc_out_b[None, :])
    return out


# speedup vs baseline: 1.0229x; 1.0229x over previous
import jax
import jax.numpy as jnp
from jax.experimental import pallas as pl
from jax.experimental.pallas import tpu as pltpu

N = 10000
F = 128
C = 64
ROWS = 400


def _fused_kernel(adj_ref, x_ref, w_in_ref, b_in_ref, b0_ref, w1_ref,
                  b1_ref, w_out_ref, b_out_ref, o_ref, t1_ref):
    p = pl.program_id(0)
    i = pl.program_id(1)
    a = adj_ref[...]

    @pl.when(p == 0)
    def _pass1():
        ax = jnp.dot(a, x_ref[...], preferred_element_type=jnp.float32)
        rs = jnp.sum(a, axis=1, keepdims=True)
        h = jnp.maximum(
            jnp.dot(ax, w_in_ref[...], preferred_element_type=jnp.float32)
            + rs * b_in_ref[...]
            + b0_ref[...],
            0.0,
        )
        t1_ref[pl.ds(i * ROWS, ROWS), :] = jnp.dot(
            h, w1_ref[...], preferred_element_type=jnp.float32
        )

    @pl.when(p == 1)
    def _pass2():
        acc = jnp.dot(a, t1_ref[...], preferred_element_type=jnp.float32)
        h = jnp.maximum(acc + b1_ref[...], 0.0)
        o_ref[...] = (
            jnp.dot(h, w_out_ref[...], preferred_element_type=jnp.float32)
            + b_out_ref[...]
        )


def kernel(x, adj, bn_gamma, bn_beta, fc_in_w, fc_in_b,
           gc0_w, gc0_b, gc1_w, gc1_b, fc_out_w, fc_out_b):
    eps = 1e-5
    scale = bn_gamma / jnp.sqrt(1.0 + eps)
    w_in = (scale[:, None] * fc_in_w.T) @ gc0_w
    b_in = ((bn_beta @ fc_in_w.T + fc_in_b) @ gc0_w)[None, :]

    grid = (2, N // ROWS)
    return pl.pallas_call(
        _fused_kernel,
        grid=grid,
        in_specs=[
            pl.BlockSpec((ROWS, N), lambda p, i: (i, 0)),
            pl.BlockSpec((N, F), lambda p, i: (0, 0)),
            pl.BlockSpec((F, F), lambda p, i: (0, 0)),
            pl.BlockSpec((1, F), lambda p, i: (0, 0)),
            pl.BlockSpec((1, F), lambda p, i: (0, 0)),
            pl.BlockSpec((F, F), lambda p, i: (0, 0)),
            pl.BlockSpec((1, F), lambda p, i: (0, 0)),
            pl.BlockSpec((F, C), lambda p, i: (0, 0)),
            pl.BlockSpec((1, C), lambda p, i: (0, 0)),
        ],
        out_specs=pl.BlockSpec((ROWS, C), lambda p, i: (i * p, 0)),
        out_shape=jax.ShapeDtypeStruct((N, C), jnp.float32),
        scratch_shapes=[pltpu.VMEM((N, F), jnp.float32)],
        compiler_params=pltpu.CompilerParams(
            dimension_semantics=("arbitrary", "arbitrary")
        ),
    )(adj, x, w_in, b_in, gc0_b, gc1_w, gc1_b, fc_out_w.T, fc_out_b[None, :])
